# baseline (speedup 1.0000x reference)
import jax
import jax.numpy as jnp
from jax import lax
from jax.experimental import pallas as pl
from jax.experimental.pallas import tpu as pltpu
from jax.experimental.pallas import tpu_sc as plsc

N = 10000
NNZ = 320000
D = 128

NC, NS, LANES = 2, 16, 16
CH = 128
ITERS = 80
CPC = NS * ITERS
RAWCPC = NNZ // NC // CH
PAD = CPC - RAWCPC
KD = D // LANES
WCH = 80
NWCH = N // WCH
WITER = (NWCH + NS - 1) // NS
NA = N + 8


def _zero_buf2d(buf, n):
    zval = jnp.zeros((LANES,), jnp.float32)

    def zrow(i, _):
        buf[i // KD, pl.ds((i % KD) * LANES, LANES)] = zval
        return 0

    lax.fori_loop(0, n * KD, zrow, 0)


def _make_sc_agg(with_deg):
    mesh = plsc.VectorSubcoreMesh(core_axis_name="c", subcore_axis_name="s")

    out_type = [jax.ShapeDtypeStruct((2 * N, D), jnp.float32)]
    scratch = [
        pltpu.VMEM((CH, D), jnp.float32),
        pltpu.VMEM((CH, D), jnp.float32),
        pltpu.VMEM((ITERS, CH), jnp.int32),
        pltpu.VMEM((CH,), jnp.int32),
        pltpu.VMEM((CH,), jnp.int32),
        pltpu.VMEM((CH,), jnp.int32),
        pltpu.VMEM((CH,), jnp.int32),
        pltpu.VMEM_SHARED((NA, D), jnp.float32),
        pltpu.SemaphoreType.DMA,
        pltpu.SemaphoreType.DMA,
        pltpu.SemaphoreType.DMA,
        pltpu.SemaphoreType.DMA,
    ]
    if with_deg:
        out_type += [jax.ShapeDtypeStruct((2 * N,), jnp.float32),
                     jax.ShapeDtypeStruct((2 * N,), jnp.float32)]
        scratch += [
            pltpu.VMEM((CH,), jnp.float32),
            pltpu.VMEM((CH,), jnp.float32),
            pltpu.VMEM((CH,), jnp.float32),
            pltpu.VMEM_SHARED((N,), jnp.float32),
            pltpu.VMEM_SHARED((NA,), jnp.float32),
            pltpu.SemaphoreType.DMA,
            pltpu.SemaphoreType.DMA,
            pltpu.SemaphoreType.DMA,
            pltpu.SemaphoreType.DMA,
            pltpu.SemaphoreType.DMA,
            pltpu.SemaphoreType.DMA,
        ]

    def body(refs):
        if with_deg:
            (table, p2d, ew, out, outd, outde,
             rows0, rows1, pall, gb0, gb1, sb0, sb1, acc,
             sg0, sg1, ss0, ss1,
             vals0, vals1, ones, accd, accde,
             se0, se1, sd0, sd1, sde0, sde1) = refs
            valss = (vals0, vals1)
            ses = (se0, se1)
            sds = (sd0, sd1)
            sdes = (sde0, sde1)
        else:
            (table, p2d, out,
             rows0, rows1, pall, gb0, gb1, sb0, sb1, acc,
             sg0, sg1, ss0, ss1) = refs
        rowss = (rows0, rows1)
        gbufs = (gb0, gb1)
        sbufs = (sb0, sb1)
        sgs = (sg0, sg1)
        sss = (ss0, ss1)

        c = lax.axis_index("c")
        s = lax.axis_index("s")

        base_row = c * CPC + s * ITERS
        pltpu.sync_copy(p2d.at[pl.ds(base_row, ITERS)], pall)

        _zero_buf2d(rows0, CH)
        if with_deg:
            zv = jnp.zeros((LANES,), jnp.float32)
            ov = jnp.ones((LANES,), jnp.float32)
            for k in range(CH // LANES):
                vals0[pl.ds(k * LANES, LANES)] = zv
                ones[pl.ds(k * LANES, LANES)] = ov
        for t in range(WITER):
            wid = t * NS + s

            @pl.when(wid < NWCH)
            def _():
                pltpu.sync_copy(rows0.at[pl.ds(0, WCH)],
                                acc.at[pl.ds(wid * WCH, WCH)])
                if with_deg:
                    pltpu.sync_copy(vals0.at[pl.ds(0, WCH)],
                                    accd.at[pl.ds(wid * WCH, WCH)])
                    pltpu.sync_copy(vals0.at[pl.ds(0, WCH)],
                                    accde.at[pl.ds(wid * WCH, WCH)])

        plsc.subcore_barrier()

        def pair(k, _):
            for sl in range(2):
                i = 2 * k + sl

                @pl.when((i >= 2) & (i <= ITERS + 1))
                def _():
                    pltpu.make_async_copy(
                        rowss[sl], acc.at[sbufs[sl]], sss[sl]).wait()
                    if with_deg:
                        pltpu.make_async_copy(
                            valss[sl], accd.at[gbufs[sl]], sds[sl]).wait()
                        pltpu.make_async_copy(
                            ones, accde.at[sbufs[sl]], sdes[sl]).wait()

                @pl.when(i < ITERS)
                def _():
                    for k16 in range(CH // LANES):
                        slc = pl.ds(k16 * LANES, LANES)
                        v = pall[i, slc]
                        gbufs[sl][slc] = v & 0xFFFF
                        sbufs[sl][slc] = v >> 16
                    pltpu.async_copy(table.at[gbufs[sl]], rowss[sl], sgs[sl])
                    if with_deg:
                        pltpu.async_copy(ew.at[sbufs[sl]], valss[sl], ses[sl])

                o = 1 - sl

                @pl.when((i >= 1) & (i <= ITERS))
                def _():
                    pltpu.make_async_copy(
                        table.at[gbufs[o]], rowss[o], sgs[o]).wait()
                    pltpu.async_copy(rowss[o], acc.at[sbufs[o]], sss[o],
                                     add=True)
                    if with_deg:
                        pltpu.make_async_copy(
                            ew.at[sbufs[o]], valss[o], ses[o]).wait()
                        pltpu.async_copy(valss[o], accd.at[gbufs[o]],
                                         sds[o], add=True)
                        pltpu.async_copy(ones, accde.at[sbufs[o]],
                                         sdes[o], add=True)

            return 0

        lax.fori_loop(0, ITERS // 2 + 1, pair, 0)
        plsc.subcore_barrier()

        for t in range(WITER):
            wid = t * NS + s

            @pl.when(wid < NWCH)
            def _():
                r0 = wid * WCH
                pltpu.sync_copy(acc.at[pl.ds(r0, WCH)], rows0.at[pl.ds(0, WCH)])
                pltpu.sync_copy(rows0.at[pl.ds(0, WCH)],
                                out.at[pl.ds(c * N + r0, WCH)])
                if with_deg:
                    pltpu.sync_copy(accd.at[pl.ds(r0, WCH)],
                                    vals0.at[pl.ds(0, WCH)])
                    pltpu.sync_copy(vals0.at[pl.ds(0, WCH)],
                                    outd.at[pl.ds(c * N + r0, WCH)])
                    pltpu.sync_copy(accde.at[pl.ds(r0, WCH)],
                                    vals0.at[pl.ds(0, WCH)])
                    pltpu.sync_copy(vals0.at[pl.ds(0, WCH)],
                                    outde.at[pl.ds(c * N + r0, WCH)])

    def wrap(*args):
        return pl.kernel(
            lambda *refs: body(refs),
            out_type=tuple(out_type) if with_deg else out_type[0],
            mesh=mesh,
            scratch_types=scratch,
        )(*args)

    return wrap


_sc_agg_deg = _make_sc_agg(True)
_sc_agg = _make_sc_agg(False)


def _pack_idx(g, s):
    p = (g | (s << 16)).reshape(2, RAWCPC, CH)
    pad = jnp.full((2, PAD, CH), N << 16, jnp.int32)
    return jnp.concatenate([p, pad], axis=1).reshape(2 * CPC, CH)


BN = 1000
RB = N // BN


def _inv(v):
    return jnp.where(v > 0, 1.0 / jnp.where(v > 0, v, 1.0), 0.0)


def _scale_body(a0_ref, a1_ref, d0_ref, d1_ref, o_ref):
    deg = d0_ref[0, 0, 0, :] + d1_ref[0, 0, 0, :]
    o_ref[...] = (a0_ref[...] + a1_ref[...]) * _inv(deg)[:, None]


def _scale(P, degp):
    return pl.pallas_call(
        _scale_body,
        grid=(RB,),
        in_specs=[pl.BlockSpec((BN, D), lambda r: (r, 0)),
                  pl.BlockSpec((BN, D), lambda r: (RB + r, 0)),
                  pl.BlockSpec((1, 1, 1, BN), lambda r: (0, r, 0, 0)),
                  pl.BlockSpec((1, 1, 1, BN), lambda r: (1, r, 0, 0))],
        out_specs=pl.BlockSpec((BN, D), lambda r: (r, 0)),
        out_shape=jax.ShapeDtypeStruct((N, D), jnp.float32),
    )(P, P, degp, degp)


def _mmrelu_body(a0_ref, a1_ref, d0_ref, d1_ref, w_ref, b_ref, o_ref):
    d = d0_ref[0, 0, 0, :] + d1_ref[0, 0, 0, :]
    v = (a0_ref[...] + a1_ref[...]) * _inv(d)[:, None]
    o_ref[...] = jnp.maximum(
        jnp.dot(v, w_ref[...], preferred_element_type=jnp.float32)
        + b_ref[0, :][None, :], 0.0)


def _mmrelu(P, dp, W, b, DO):
    cb = DO // 128
    return pl.pallas_call(
        _mmrelu_body,
        grid=(cb, RB),
        in_specs=[pl.BlockSpec((BN, D), lambda c, r: (r, 0)),
                  pl.BlockSpec((BN, D), lambda c, r: (RB + r, 0)),
                  pl.BlockSpec((1, 1, 1, BN), lambda c, r: (0, r, 0, 0)),
                  pl.BlockSpec((1, 1, 1, BN), lambda c, r: (1, r, 0, 0)),
                  pl.BlockSpec((D, 128), lambda c, r: (0, c)),
                  pl.BlockSpec((1, 128), lambda c, r: (0, c))],
        out_specs=pl.BlockSpec((BN, 128), lambda c, r: (r, c)),
        out_shape=jax.ShapeDtypeStruct((N, DO), jnp.float32),
    )(P, P, dp, dp, W, b.reshape(1, DO))


def kernel(x, edge_index, edge_weight, batch, W1, b1, W2, b2):
    row = edge_index[0].astype(jnp.int32)
    col = edge_index[1].astype(jnp.int32)
    ew = edge_weight.astype(jnp.float32)

    pA = _pack_idx(row, col)
    pB = _pack_idx(col, row)
    ew_p = jnp.concatenate([ew, jnp.zeros((8,), jnp.float32)])

    P1, dpart, depart = _sc_agg_deg(x, pA, ew_p)
    dp = dpart.reshape(2, RB, 1, BN)
    dep = depart.reshape(2, RB, 1, BN)
    T2 = _scale(P1, dep)
    P2 = _sc_agg(T2, pB)
    h1 = _mmrelu(P2, dp, W1, b1, D)

    P3 = _sc_agg(h1, pA)
    T4 = _scale(P3, dep)
    P4 = _sc_agg(T4, pB)
    return _mmrelu(P4, dp, W2, b2, 2 * D)
